# baseline (speedup 1.0000x reference)
import functools

import jax
import jax.numpy as jnp
from jax import lax
from jax.experimental import pallas as pl
from jax.experimental.pallas import tpu as pltpu
from jax.experimental.pallas import tpu_sc as plsc

B = 4096
L = 200
D = 32
OUT_DIM = 20

NC = 2
NS = 16
NW = NC * NS
BPW = B // NW
C0 = 104
C1 = L - C0
NBUF = 4

_mesh = plsc.VectorSubcoreMesh(core_axis_name="c", subcore_axis_name="s")


@functools.partial(
    pl.kernel,
    out_type=jax.ShapeDtypeStruct((B, D), jnp.float32),
    mesh=_mesh,
    scratch_types=[
        pltpu.VMEM((BPW, L), jnp.int32),
        pltpu.VMEM((BPW, D), jnp.float32),
        pltpu.VMEM((NBUF, L, D), jnp.float32),
        [pltpu.SemaphoreType.DMA] * NBUF,
    ],
    compiler_params=pltpu.CompilerParams(use_tc_tiling_on_sc=False),
)
def _pool(data_hbm, table_hbm, out_hbm, idx_v, out_v, rows_v, sems):
    wid = lax.axis_index("s") * NC + lax.axis_index("c")
    base = wid * BPW

    pltpu.sync_copy(data_hbm.at[pl.ds(base, BPW)], idx_v)

    def fire(i, slot):
        rows = rows_v.at[slot]
        pltpu.async_copy(
            table_hbm.at[idx_v.at[i, pl.ds(0, C0)]], rows.at[pl.ds(0, C0)],
            sems[slot],
        )
        pltpu.async_copy(
            table_hbm.at[idx_v.at[i, pl.ds(C0, C1)]], rows.at[pl.ds(C0, C1)],
            sems[slot],
        )

    def drain(i, slot):
        rows = rows_v.at[slot]
        pltpu.make_async_copy(
            table_hbm.at[idx_v.at[i, pl.ds(0, C0)]], rows.at[pl.ds(0, C0)],
            sems[slot],
        ).wait()
        pltpu.make_async_copy(
            table_hbm.at[idx_v.at[i, pl.ds(C0, C1)]], rows.at[pl.ds(C0, C1)],
            sems[slot],
        ).wait()

    def accumulate(i, slot):
        rows = rows_v.at[slot]
        z = jnp.zeros((16,), jnp.float32)
        a = [z] * 4
        bb = [z] * 4
        for l in range(L):
            c = l % 4
            a[c] = a[c] + rows[l, pl.ds(0, 16)]
            bb[c] = bb[c] + rows[l, pl.ds(16, 16)]
        out_v[i, pl.ds(0, 16)] = (a[0] + a[1]) + (a[2] + a[3])
        out_v[i, pl.ds(16, 16)] = (bb[0] + bb[1]) + (bb[2] + bb[3])

    for s in range(NBUF):
        fire(s, s)

    def loop_body(j, _):
        i = j * NBUF
        for s in range(NBUF):
            drain(i + s, s)
            accumulate(i + s, s)

            @pl.when(i + s + NBUF < BPW)
            def _refire(i=i, s=s):
                fire(i + s + NBUF, s)

        return 0

    lax.fori_loop(0, BPW // NBUF, loop_body, 0)

    pltpu.sync_copy(out_v, out_hbm.at[pl.ds(base, BPW)])


def _linear_body(pooled_ref, len_ref, w_ref, b_ref, out_ref):
    x = pooled_ref[...] / len_ref[...].astype(jnp.float32)
    out_ref[...] = (
        lax.dot_general(
            x, w_ref[...], (((1,), (1,)), ((), ())),
            preferred_element_type=jnp.float32,
        )
        + b_ref[...]
    )


_linear = pl.pallas_call(
    _linear_body,
    out_shape=jax.ShapeDtypeStruct((B, OUT_DIM), jnp.float32),
)


def kernel(data, length, table, W, b):
    data = data.astype(jnp.int32)
    pooled = _pool(data, table)
    return _linear(pooled, length.reshape(B, 1), W, b.reshape(1, OUT_DIM))
